# Initial kernel scaffold; baseline (speedup 1.0000x reference)
import jax
import jax.numpy as jnp
from jax.experimental import pallas as pl


def kernel(xy_pos, x_table, y_table):
    raise NotImplementedError("write your pallas kernel here")


# speedup vs baseline: 1.6319x; 1.6319x over previous
import functools

import jax
import jax.numpy as jnp
from jax import lax
from jax.experimental import pallas as pl
from jax.experimental.pallas import tpu as pltpu
from jax.experimental.pallas import tpu_sc as plsc

_SCALE = 100.0
_LANES = 16

_info = plsc.get_sparse_core_info()
_NC = _info.num_cores
_NS = _info.num_subcores
_NW = _NC * _NS


def _round_nearest_even(x):
    t = x.astype(jnp.int32)
    f = x - t.astype(jnp.float32)
    odd = (t & 1) == 1
    up = (f > 0.5) | ((f == 0.5) & odd)
    return jnp.where(up, t + 1, t)


@functools.lru_cache(maxsize=None)
def _make_sc_lookup(batch, dim):
    bpw = batch // _NW
    nvec = bpw // _LANES
    mesh = plsc.VectorSubcoreMesh(core_axis_name="c", subcore_axis_name="s")

    @functools.partial(
        pl.kernel,
        mesh=mesh,
        out_type=jax.ShapeDtypeStruct((batch, 2 * dim), jnp.float32),
        compiler_params=pltpu.CompilerParams(use_tc_tiling_on_sc=False),
        scratch_types=[
            pltpu.VMEM((bpw,), jnp.float32),
            pltpu.VMEM((bpw,), jnp.float32),
            pltpu.VMEM((bpw,), jnp.int32),
            pltpu.VMEM((bpw,), jnp.int32),
            pltpu.VMEM((bpw, dim), jnp.float32),
            pltpu.VMEM((bpw, dim), jnp.float32),
            pltpu.SemaphoreType.DMA,
        ],
    )
    def lookup(xpos_hbm, ypos_hbm, xtab_hbm, ytab_hbm, out_hbm,
               xpos_v, ypos_v, xidx_v, yidx_v, xrows_v, yrows_v, sem):
        wid = lax.axis_index("s") * _NC + lax.axis_index("c")
        base = wid * bpw
        pltpu.sync_copy(xpos_hbm.at[pl.ds(base, bpw)], xpos_v)
        pltpu.sync_copy(ypos_hbm.at[pl.ds(base, bpw)], ypos_v)

        def body(i, carry):
            sl = pl.ds(i * _LANES, _LANES)
            xidx_v[sl] = _round_nearest_even(xpos_v[sl] * _SCALE)
            yidx_v[sl] = _round_nearest_even(ypos_v[sl] * _SCALE)
            return carry

        lax.fori_loop(0, nvec, body, 0)

        cx = pltpu.async_copy(xtab_hbm.at[xidx_v], xrows_v, sem)
        cy = pltpu.async_copy(ytab_hbm.at[yidx_v], yrows_v, sem)
        cx.wait()
        cy.wait()

        pltpu.sync_copy(xrows_v, out_hbm.at[pl.ds(base, bpw), pl.ds(0, dim)])
        pltpu.sync_copy(yrows_v, out_hbm.at[pl.ds(base, bpw), pl.ds(dim, dim)])

    return lookup


def kernel(xy_pos, x_table, y_table):
    batch = xy_pos.shape[0]
    dim = x_table.shape[1]
    xpos = xy_pos[:, 0]
    ypos = xy_pos[:, 1]
    return _make_sc_lookup(batch, dim)(xpos, ypos, x_table, y_table)
